# baseline (speedup 1.0000x reference)
import functools

import jax
import jax.numpy as jnp
from jax import lax
from jax.experimental import pallas as pl
from jax.experimental.pallas import tpu as pltpu
from jax.experimental.pallas import tpu_sc as plsc

N = 50000
E = 800000
H = 4
D_HID = 16
N_CLASSES = 40
NEG_SLOPE = 0.2

BN = 256
NBLK = 196
NPAD = BN * NBLK
NW = 32
WIN = 128
SROWS = 8
WROWS = 200
ROWS = NW * WROWS
EPAD = ROWS * WIN
NSTAGE = WROWS // SROWS
MROWS = ROWS // 16
MSROWS = 8
MSTAGE = MROWS // MSROWS

_mesh = plsc.VectorSubcoreMesh(core_axis_name="c", subcore_axis_name="s")
_SC_PARAMS = pltpu.CompilerParams(use_tc_tiling_on_sc=False)


def _lr(x):
    return jnp.maximum(x, NEG_SLOPE * x)


def _elu(x):
    return jnp.where(x > 0, x, jnp.exp(jnp.minimum(x, 0.0)) - 1.0)


def _tile16(x):
    return jnp.concatenate([x, x, x, x], axis=1)


def _hist_body(dst2d, zeros_n, out_hbm, dbuf, ones_v, ssem, acc):
    cid = lax.axis_index("c")
    sid = lax.axis_index("s")
    r0 = WROWS * (cid * 16 + sid)

    for v in range(8):
        ones_v[pl.ds(16 * v, 16)] = jnp.full((16,), 1.0, jnp.float32)

    @pl.when(sid == 0)
    def _():
        pltpu.sync_copy(zeros_n, acc)
    plsc.subcore_barrier()

    def stage_body(j, carry):
        pltpu.sync_copy(dst2d.at[pl.ds(r0 + SROWS * j, SROWS)], dbuf)
        last = None
        for k in range(SROWS):
            if last is not None:
                last.wait()
            last = pltpu.async_copy(ones_v, acc.at[dbuf.at[k]], ssem,
                                    add=True)
        last.wait()
        return carry

    lax.fori_loop(0, NSTAGE, stage_body, 0)

    plsc.subcore_barrier()

    @pl.when(sid == 0)
    def _():
        pltpu.sync_copy(acc, out_hbm.at[cid])


_hist = functools.partial(
    pl.kernel,
    out_type=jax.ShapeDtypeStruct((2, NPAD), jnp.float32),
    mesh=_mesh,
    compiler_params=_SC_PARAMS,
    scratch_types=[
        pltpu.VMEM((SROWS, WIN), jnp.int32),
        pltpu.VMEM((WIN,), jnp.float32),
        pltpu.SemaphoreType.DMA,
        pltpu.VMEM_SHARED((NPAD,), jnp.float32),
    ],
)(_hist_body)


def _att0_body(src2d, dst2d, deg16, consts, zeros_n16, s_out,
               sbuf, dbuf, gs, gd, wbuf, cvm, gsem, ssem, acc_s):
    cid = lax.axis_index("c")
    sid = lax.axis_index("s")
    r0 = WROWS * (cid * 16 + sid)

    pltpu.sync_copy(consts, cvm)
    clv = cvm[0, :]
    crv = cvm[1, :]
    m0v = cvm[2, :]
    iota = lax.iota(jnp.int32, 16)
    tmask = ((iota >> 2) & 1) == 1

    @pl.when(sid == 0)
    def _():
        pltpu.sync_copy(zeros_n16, acc_s)
    plsc.subcore_barrier()

    def stage_body(j, carry):
        pltpu.sync_copy(src2d.at[pl.ds(r0 + SROWS * j, SROWS)], sbuf)
        pltpu.sync_copy(dst2d.at[pl.ds(r0 + SROWS * j, SROWS)], dbuf)
        gds = [(pltpu.async_copy(deg16.at[sbuf.at[k]], gs.at[k], gsem),
                pltpu.async_copy(deg16.at[dbuf.at[k]], gd.at[k], gsem))
               for k in range(SROWS)]
        last = None
        for k in range(SROWS):
            gds[k][0].wait()
            gds[k][1].wait()

            def row_block(rb, c2):
                for q in range(4):
                    r = 4 * rb + q
                    ds_ = gs[k, r, :]
                    dd_ = gd[k, r, :]
                    u = ds_ * clv + dd_ * crv
                    w = jnp.exp(jnp.maximum(u, NEG_SLOPE * u) - m0v)
                    wbuf[k, r, :] = jnp.where(tmask, w * ds_, w)
                return c2

            lax.fori_loop(0, 32, row_block, 0)
            if last is not None:
                last.wait()
            last = pltpu.async_copy(
                wbuf.at[k], acc_s.at[dbuf.at[k]], ssem, add=True)
        last.wait()
        return carry

    lax.fori_loop(0, NSTAGE, stage_body, 0)

    plsc.subcore_barrier()

    @pl.when(sid == 0)
    def _():
        pltpu.sync_copy(acc_s, s_out.at[cid])


_att0 = functools.partial(
    pl.kernel,
    out_type=jax.ShapeDtypeStruct((2, NPAD, 16), jnp.float32),
    mesh=_mesh,
    compiler_params=_SC_PARAMS,
    scratch_types=[
        pltpu.VMEM((SROWS, WIN), jnp.int32),
        pltpu.VMEM((SROWS, WIN), jnp.int32),
        pltpu.VMEM((SROWS, WIN, 16), jnp.float32),
        pltpu.VMEM((SROWS, WIN, 16), jnp.float32),
        pltpu.VMEM((SROWS, WIN, 16), jnp.float32),
        pltpu.VMEM((3, 16), jnp.float32),
        pltpu.SemaphoreType.DMA,
        pltpu.SemaphoreType.DMA,
        pltpu.VMEM_SHARED((NPAD, 16), jnp.float32),
    ],
)(_att0_body)


def _att_body(src2d, dst2d, el, er, mconst, zeros_n16, w_out, s_out,
              sbuf, dbuf, ges, ger, wbuf, wobuf, cvm, gsem, ssem, wsem,
              acc_s):
    cid = lax.axis_index("c")
    sid = lax.axis_index("s")
    r0 = WROWS * (cid * 16 + sid)

    pltpu.sync_copy(mconst, cvm)
    mv = cvm[0, :]

    @pl.when(sid == 0)
    def _():
        pltpu.sync_copy(zeros_n16, acc_s)
    plsc.subcore_barrier()

    def stage_body(j, carry):
        rr = r0 + SROWS * j
        pltpu.sync_copy(src2d.at[pl.ds(rr, SROWS)], sbuf)
        pltpu.sync_copy(dst2d.at[pl.ds(rr, SROWS)], dbuf)
        gds = [(pltpu.async_copy(el.at[sbuf.at[k]], ges.at[k], gsem),
                pltpu.async_copy(er.at[dbuf.at[k]], ger.at[k], gsem))
               for k in range(SROWS)]
        last = None
        wds = []
        for k in range(SROWS):
            gds[k][0].wait()
            gds[k][1].wait()

            def row_block(rb, c2):
                for q in range(4):
                    r = 4 * rb + q
                    u = ges[k, r, :] + ger[k, r, :]
                    w = jnp.exp(jnp.maximum(u, NEG_SLOPE * u) - mv)
                    wbuf[k, r, :] = w
                    wobuf[k, pl.ds(16 * r, 16)] = w
                return c2

            lax.fori_loop(0, 32, row_block, 0)
            if last is not None:
                last.wait()
            last = pltpu.async_copy(
                wbuf.at[k], acc_s.at[dbuf.at[k]], ssem, add=True)
            wds.append(pltpu.async_copy(
                wobuf.at[k], w_out.at[pl.ds((rr + k) * 2048, 2048)], wsem))
        last.wait()
        for d in wds:
            d.wait()
        return carry

    lax.fori_loop(0, NSTAGE, stage_body, 0)

    plsc.subcore_barrier()

    @pl.when(sid == 0)
    def _():
        pltpu.sync_copy(acc_s, s_out.at[cid])


_att = functools.partial(
    pl.kernel,
    out_type=(jax.ShapeDtypeStruct((EPAD * 16,), jnp.float32),
              jax.ShapeDtypeStruct((2, NPAD, 16), jnp.float32)),
    mesh=_mesh,
    compiler_params=_SC_PARAMS,
    scratch_types=[
        pltpu.VMEM((SROWS, WIN), jnp.int32),
        pltpu.VMEM((SROWS, WIN), jnp.int32),
        pltpu.VMEM((SROWS, WIN, 16), jnp.float32),
        pltpu.VMEM((SROWS, WIN, 16), jnp.float32),
        pltpu.VMEM((SROWS, WIN, 16), jnp.float32),
        pltpu.VMEM((SROWS, WIN * 16), jnp.float32),
        pltpu.VMEM((1, 16), jnp.float32),
        pltpu.SemaphoreType.DMA,
        pltpu.SemaphoreType.DMA,
        pltpu.SemaphoreType.DMA,
        pltpu.VMEM_SHARED((NPAD, 16), jnp.float32),
    ],
)(_att_body)


def _msg_body(src2d, dst2d, w_hbm, f0t, f1t, f2t, f3t, zeros_n16, out_hbm,
              sbuf, dbuf, wstg, fbuf, mbuf, gsem, ssem, psem, acc):
    cid = lax.axis_index("c")
    sid = lax.axis_index("s")
    r0 = MROWS * sid

    def sweep(h, ftab):
        @pl.when(sid == 0)
        def _():
            pltpu.sync_copy(zeros_n16, acc)
        plsc.subcore_barrier()

        pltpu.async_copy(src2d.at[pl.ds(r0, MSROWS)], sbuf.at[0], psem)
        pltpu.async_copy(dst2d.at[pl.ds(r0, MSROWS)], dbuf.at[0], psem)
        pltpu.async_copy(w_hbm.at[pl.ds(r0 * 2048, MSROWS * 2048)],
                         wstg.at[0], psem)

        def stage_body(j, carry):
            p = j & 1
            rr = r0 + MSROWS * j
            pltpu.make_async_copy(
                src2d.at[pl.ds(rr, MSROWS)], sbuf.at[p], psem).wait()
            pltpu.make_async_copy(
                dst2d.at[pl.ds(rr, MSROWS)], dbuf.at[p], psem).wait()
            pltpu.make_async_copy(
                w_hbm.at[pl.ds(rr * 2048, MSROWS * 2048)], wstg.at[p],
                psem).wait()
            rn = jnp.minimum(rr + MSROWS, r0 + MROWS - MSROWS)
            pltpu.async_copy(src2d.at[pl.ds(rn, MSROWS)], sbuf.at[1 - p],
                             psem)
            pltpu.async_copy(dst2d.at[pl.ds(rn, MSROWS)], dbuf.at[1 - p],
                             psem)
            pltpu.async_copy(w_hbm.at[pl.ds(rn * 2048, MSROWS * 2048)],
                             wstg.at[1 - p], psem)
            gds = [pltpu.async_copy(ftab.at[sbuf.at[p, k]], fbuf.at[k], gsem)
                   for k in range(MSROWS)]
            last = None
            for k in range(MSROWS):
                gds[k].wait()

                def row_block(rb, c2):
                    for q in range(8):
                        r = 8 * rb + q
                        wv = wstg[p, pl.ds(2048 * k + 16 * r, 16)]
                        mbuf[k, r, :] = fbuf[k, r, :] * jnp.broadcast_to(
                            wv[h], (16,))
                    return c2

                lax.fori_loop(0, 16, row_block, 0)
                if last is not None:
                    last.wait()
                last = pltpu.async_copy(
                    mbuf.at[k], acc.at[dbuf.at[p, k]], ssem, add=True)
            last.wait()
            return carry

        lax.fori_loop(0, MSTAGE, stage_body, 0)
        pltpu.make_async_copy(
            src2d.at[pl.ds(r0, MSROWS)], sbuf.at[0], psem).wait()
        pltpu.make_async_copy(
            dst2d.at[pl.ds(r0, MSROWS)], dbuf.at[0], psem).wait()
        pltpu.make_async_copy(
            w_hbm.at[pl.ds(r0 * 2048, MSROWS * 2048)], wstg.at[0],
            psem).wait()
        plsc.subcore_barrier()

        @pl.when(sid == 0)
        def _():
            pltpu.sync_copy(acc, out_hbm.at[h])
        plsc.subcore_barrier()

    @pl.when(cid == 0)
    def _():
        sweep(0, f0t)
        sweep(1, f1t)

    @pl.when(cid == 1)
    def _():
        sweep(2, f2t)
        sweep(3, f3t)


_msg = functools.partial(
    pl.kernel,
    out_type=jax.ShapeDtypeStruct((4, NPAD, 16), jnp.float32),
    mesh=_mesh,
    compiler_params=_SC_PARAMS,
    scratch_types=[
        pltpu.VMEM((2, MSROWS, WIN), jnp.int32),
        pltpu.VMEM((2, MSROWS, WIN), jnp.int32),
        pltpu.VMEM((2, MSROWS * 2048), jnp.float32),
        pltpu.VMEM((MSROWS, WIN, 16), jnp.float32),
        pltpu.VMEM((MSROWS, WIN, 16), jnp.float32),
        pltpu.SemaphoreType.DMA,
        pltpu.SemaphoreType.DMA,
        pltpu.SemaphoreType.DMA,
        pltpu.VMEM_SHARED((NPAD, 16), jnp.float32),
    ],
)(_msg_body)


def _g_body(src2d, dst2d, w_hbm, s2t, zeros_n16, g_out,
            sbuf, dbuf, wstg, rbuf, abuf, gsem, ssem, acc_g):
    cid = lax.axis_index("c")
    sid = lax.axis_index("s")
    r0 = WROWS * (cid * 16 + sid)

    @pl.when(sid == 0)
    def _():
        pltpu.sync_copy(zeros_n16, acc_g)
    plsc.subcore_barrier()

    zero16 = jnp.zeros((16,), jnp.float32)

    def stage_body(j, carry):
        rr = r0 + SROWS * j
        pltpu.sync_copy(src2d.at[pl.ds(rr, SROWS)], sbuf)
        pltpu.sync_copy(dst2d.at[pl.ds(rr, SROWS)], dbuf)
        wd = pltpu.async_copy(
            w_hbm.at[pl.ds(rr * 2048, SROWS * 2048)], wstg, gsem)
        gds = [pltpu.async_copy(s2t.at[dbuf.at[k]], rbuf.at[k], gsem)
               for k in range(SROWS)]
        wd.wait()
        last = None
        for k in range(SROWS):
            gds[k].wait()

            def row_block(rb, c2):
                for q in range(4):
                    r = 4 * rb + q
                    sv = rbuf[k, r, :]
                    wv = wstg[pl.ds(2048 * k + 16 * r, 16)]
                    a = jnp.where(sv > 0, wv / jnp.maximum(sv, 1e-30),
                                  zero16)
                    abuf[k, r, :] = a
                return c2

            lax.fori_loop(0, 32, row_block, 0)
            if last is not None:
                last.wait()
            last = pltpu.async_copy(
                abuf.at[k], acc_g.at[sbuf.at[k]], ssem, add=True)
        last.wait()
        return carry

    lax.fori_loop(0, NSTAGE, stage_body, 0)

    plsc.subcore_barrier()

    @pl.when(sid == 0)
    def _():
        pltpu.sync_copy(acc_g, g_out.at[cid])


_g_pass = functools.partial(
    pl.kernel,
    out_type=jax.ShapeDtypeStruct((2, NPAD, 16), jnp.float32),
    mesh=_mesh,
    compiler_params=_SC_PARAMS,
    scratch_types=[
        pltpu.VMEM((SROWS, WIN), jnp.int32),
        pltpu.VMEM((SROWS, WIN), jnp.int32),
        pltpu.VMEM((SROWS * 2048,), jnp.float32),
        pltpu.VMEM((SROWS, WIN, 16), jnp.float32),
        pltpu.VMEM((SROWS, WIN, 16), jnp.float32),
        pltpu.SemaphoreType.DMA,
        pltpu.SemaphoreType.DMA,
        pltpu.VMEM_SHARED((NPAD, 16), jnp.float32),
    ],
)(_g_body)


def _t0_body(p_ref, deg16_ref, mm_ref):
    b = pl.program_id(0)
    deg = p_ref[0, :] + p_ref[1, :]
    deg16_ref[...] = jnp.broadcast_to(deg[:, None], (BN, 16))
    rows = b * BN + lax.iota(jnp.int32, BN)
    dmax = jnp.max(jnp.where(rows < N, deg, -jnp.inf))
    dmin = jnp.min(jnp.where(rows < N, deg, jnp.inf))

    @pl.when(b == 0)
    def _():
        mm_ref[...] = jnp.full((2, 128), 0.0, jnp.float32)
    mm_ref[0:1, :] = jnp.maximum(mm_ref[0:1, :], dmax)
    mm_ref[1:2, :] = jnp.minimum(mm_ref[1:2, :], dmin)


def _t0(degp):
    return pl.pallas_call(
        _t0_body,
        grid=(NBLK,),
        in_specs=[pl.BlockSpec((2, BN), lambda b: (0, b))],
        out_specs=[pl.BlockSpec((BN, 16), lambda b: (b, 0)),
                   pl.BlockSpec((2, 128), lambda b: (0, 0))],
        out_shape=[jax.ShapeDtypeStruct((NPAD, 16), jnp.float32),
                   jax.ShapeDtypeStruct((2, 128), jnp.float32)],
    )(degp)


def _t1_body(sp_ref, w0h_ref, w1c_ref, f0_ref, f1_ref, f2_ref, f3_ref,
             el_ref, er_ref, mm_ref):
    b = pl.program_id(0)
    s0 = sp_ref[0, :, 0:4] + sp_ref[1, :, 0:4]
    t0 = sp_ref[0, :, 4:8] + sp_ref[1, :, 4:8]
    S0 = jnp.where(s0 > 0, t0 / jnp.maximum(s0, 1e-30), 0.0)
    h1 = _elu(jnp.concatenate(
        [S0[:, h:h + 1] * w0h_ref[h:h + 1, :] for h in range(4)], axis=1))
    fe = jnp.dot(h1, w1c_ref[...], preferred_element_type=jnp.float32)
    f0_ref[...] = fe[:, 0:16]
    f1_ref[...] = fe[:, 16:32]
    f2_ref[...] = fe[:, 32:48]
    f3_ref[...] = fe[:, 48:64]
    el = fe[:, 64:68]
    er = fe[:, 68:72]
    el_ref[...] = _tile16(el)
    er_ref[...] = _tile16(er)
    mask = lax.broadcasted_iota(jnp.int32, (BN, 4), 0) + b * BN < N
    elm = jnp.max(jnp.where(mask, el, -jnp.inf), axis=0, keepdims=True)
    erm = jnp.max(jnp.where(mask, er, -jnp.inf), axis=0, keepdims=True)

    @pl.when(b == 0)
    def _():
        mm_ref[...] = jnp.full((2, 4), -jnp.inf, jnp.float32)
    mm_ref[0:1, :] = jnp.maximum(mm_ref[0:1, :], elm)
    mm_ref[1:2, :] = jnp.maximum(mm_ref[1:2, :], erm)


def _t1(sp, w0h, w1c):
    nspec = pl.BlockSpec((BN, 16), lambda b: (b, 0))
    nshape = jax.ShapeDtypeStruct((NPAD, 16), jnp.float32)
    return pl.pallas_call(
        _t1_body,
        grid=(NBLK,),
        in_specs=[pl.BlockSpec((2, BN, 16), lambda b: (0, b, 0)),
                  pl.BlockSpec((4, 16), lambda b: (0, 0)),
                  pl.BlockSpec((64, 72), lambda b: (0, 0))],
        out_specs=[nspec, nspec, nspec, nspec, nspec, nspec,
                   pl.BlockSpec((2, 4), lambda b: (0, 0))],
        out_shape=[nshape, nshape, nshape, nshape, nshape, nshape,
                   jax.ShapeDtypeStruct((2, 4), jnp.float32)],
    )(sp, w0h, w1c)


def _t2_body(o_ref, s_ref, b2_ref, h2_ref, el_ref, er_ref, mm_ref):
    b = pl.program_id(0)
    s1 = s_ref[0, :, 0:4] + s_ref[1, :, 0:4]
    parts = []
    for h in range(4):
        oh = o_ref[h]
        sh = s1[:, h:h + 1]
        parts.append(jnp.where(sh > 0, oh / jnp.maximum(sh, 1e-30), 0.0))
    h2 = _elu(jnp.concatenate(parts, axis=1))
    h2_ref[...] = h2
    ee = jnp.dot(h2, b2_ref[...], preferred_element_type=jnp.float32)
    el = ee[:, 0:4]
    er = ee[:, 4:8]
    el_ref[...] = _tile16(el)
    er_ref[...] = _tile16(er)
    mask = lax.broadcasted_iota(jnp.int32, (BN, 4), 0) + b * BN < N
    elm = jnp.max(jnp.where(mask, el, -jnp.inf), axis=0, keepdims=True)
    erm = jnp.max(jnp.where(mask, er, -jnp.inf), axis=0, keepdims=True)

    @pl.when(b == 0)
    def _():
        mm_ref[...] = jnp.full((2, 4), -jnp.inf, jnp.float32)
    mm_ref[0:1, :] = jnp.maximum(mm_ref[0:1, :], elm)
    mm_ref[1:2, :] = jnp.maximum(mm_ref[1:2, :], erm)


def _t2(out1, s1p, b2):
    return pl.pallas_call(
        _t2_body,
        grid=(NBLK,),
        in_specs=[pl.BlockSpec((4, BN, 16), lambda b: (0, b, 0)),
                  pl.BlockSpec((2, BN, 16), lambda b: (0, b, 0)),
                  pl.BlockSpec((64, 8), lambda b: (0, 0))],
        out_specs=[pl.BlockSpec((BN, 64), lambda b: (b, 0)),
                   pl.BlockSpec((BN, 16), lambda b: (b, 0)),
                   pl.BlockSpec((BN, 16), lambda b: (b, 0)),
                   pl.BlockSpec((2, 4), lambda b: (0, 0))],
        out_shape=[jax.ShapeDtypeStruct((NPAD, 64), jnp.float32),
                   jax.ShapeDtypeStruct((NPAD, 16), jnp.float32),
                   jax.ShapeDtypeStruct((NPAD, 16), jnp.float32),
                   jax.ShapeDtypeStruct((2, 4), jnp.float32)],
    )(out1, s1p, b2)


def _ts_body(s_ref, st_ref):
    s2 = s_ref[0, :, 0:4] + s_ref[1, :, 0:4]
    st_ref[...] = _tile16(s2)


def _ts(s2p):
    return pl.pallas_call(
        _ts_body,
        grid=(NBLK,),
        in_specs=[pl.BlockSpec((2, BN, 16), lambda b: (0, b, 0))],
        out_specs=pl.BlockSpec((BN, 16), lambda b: (b, 0)),
        out_shape=jax.ShapeDtypeStruct((NPAD, 16), jnp.float32),
    )(s2p)


def _t3_body(g_ref, h2_ref, G_ref):
    b = pl.program_id(0)
    g = g_ref[0, :, 0:4] + g_ref[1, :, 0:4]
    mask = lax.broadcasted_iota(jnp.int32, (BN, 4), 0) + b * BN < N
    g = jnp.where(mask, g, 0.0)
    Gb = lax.dot_general(g, h2_ref[...], (((0,), (0,)), ((), ())),
                         preferred_element_type=jnp.float32)

    @pl.when(b == 0)
    def _():
        G_ref[...] = jnp.zeros((4, 64), jnp.float32)
    G_ref[...] += Gb


def _t3(gp, h2):
    return pl.pallas_call(
        _t3_body,
        grid=(NBLK,),
        in_specs=[pl.BlockSpec((2, BN, 16), lambda b: (0, b, 0)),
                  pl.BlockSpec((BN, 64), lambda b: (b, 0))],
        out_specs=pl.BlockSpec((4, 64), lambda b: (0, 0)),
        out_shape=jax.ShapeDtypeStruct((4, 64), jnp.float32),
    )(gp, h2)


def _pad_edges(idx):
    return jnp.concatenate(
        [idx, jnp.full((EPAD - E,), N, jnp.int32)]).reshape(ROWS, WIN)


def kernel(edge_index, W0, a_l0, a_r0, W1, a_l1, a_r1, W2, a_l2, a_r2):
    src2d = _pad_edges(edge_index[0].astype(jnp.int32))
    dst2d = _pad_edges(edge_index[1].astype(jnp.int32))
    zeros_n = jnp.zeros((NPAD,), jnp.float32)
    zeros_n16 = jnp.zeros((NPAD, 16), jnp.float32)

    degp = _hist(dst2d, zeros_n)
    deg16, dmm = _t0(degp)
    dmax, dmin = dmm[0, 0], dmm[1, 0]

    W0h = W0.reshape(H, D_HID)
    cl0 = jnp.sum(W0h * a_l0, axis=1)
    cr0 = jnp.sum(W0h * a_r0, axis=1)
    mel0 = jnp.maximum(cl0 * dmax, cl0 * dmin)
    mer0 = jnp.maximum(cr0 * dmax, cr0 * dmin)
    mh0 = _lr(mel0 + mer0)
    consts0 = jnp.stack([jnp.tile(cl0, 4), jnp.tile(cr0, 4),
                         jnp.tile(mh0, 4)])
    sp0 = _att0(src2d, dst2d, deg16, consts0, zeros_n16)

    W1r = W1.reshape(64, H, D_HID)
    bl1 = jnp.einsum("khd,hd->kh", W1r, a_l1)
    br1 = jnp.einsum("khd,hd->kh", W1r, a_r1)
    w1c = jnp.concatenate([W1, bl1, br1], axis=1)
    f0t, f1t, f2t, f3t, el1, er1, mm1 = _t1(sp0, W0h, w1c)
    mh1 = _lr(mm1[0] + mm1[1])
    w1, s1p = _att(src2d, dst2d, el1, er1,
                   jnp.tile(mh1, 4)[None, :], zeros_n16)
    out1 = _msg(src2d, dst2d, w1, f0t, f1t, f2t, f3t,
                zeros_n16)

    W2r = W2.reshape(64, H, N_CLASSES)
    bl2 = jnp.einsum("khc,hc->kh", W2r, a_l2)
    br2 = jnp.einsum("khc,hc->kh", W2r, a_r2)
    b2 = jnp.concatenate([bl2, br2], axis=1)
    h2, el2, er2, mm2 = _t2(out1, s1p, b2)
    mh2 = _lr(mm2[0] + mm2[1])
    w2, s2p = _att(src2d, dst2d, el2, er2,
                   jnp.tile(mh2, 4)[None, :], zeros_n16)
    s2t = _ts(s2p)
    gp = _g_pass(src2d, dst2d, w2, s2t, zeros_n16)

    G = _t3(gp, h2)
    hg = jnp.einsum("hk,khc->c", G, W2r) / (N * H)
    return hg[None, :]


# speedup vs baseline: 1.0443x; 1.0443x over previous
import functools

import jax
import jax.numpy as jnp
from jax import lax
from jax.experimental import pallas as pl
from jax.experimental.pallas import tpu as pltpu
from jax.experimental.pallas import tpu_sc as plsc

N = 50000
E = 800000
H = 4
D_HID = 16
N_CLASSES = 40
NEG_SLOPE = 0.2

BN = 256
NBLK = 196
NPAD = BN * NBLK
NW = 32
WIN = 128
SROWS = 8
WROWS = 200
ROWS = NW * WROWS
EPAD = ROWS * WIN
NSTAGE = WROWS // SROWS
MROWS = ROWS // 16
MSROWS = 8
MSTAGE = MROWS // MSROWS

_mesh = plsc.VectorSubcoreMesh(core_axis_name="c", subcore_axis_name="s")
_SC_PARAMS = pltpu.CompilerParams(use_tc_tiling_on_sc=False)


def _lr(x):
    return jnp.maximum(x, NEG_SLOPE * x)


def _elu(x):
    return jnp.where(x > 0, x, jnp.exp(jnp.minimum(x, 0.0)) - 1.0)


def _tile16(x):
    return jnp.concatenate([x, x, x, x], axis=1)


def _hist_body(dst2d, zeros_n, out_hbm, dbuf, ones_v, ssem, acc):
    cid = lax.axis_index("c")
    sid = lax.axis_index("s")
    r0 = WROWS * (cid * 16 + sid)

    for v in range(8):
        ones_v[pl.ds(16 * v, 16)] = jnp.full((16,), 1.0, jnp.float32)

    @pl.when(sid == 0)
    def _():
        pltpu.sync_copy(zeros_n, acc)
    plsc.subcore_barrier()

    def stage_body(j, carry):
        pltpu.sync_copy(dst2d.at[pl.ds(r0 + SROWS * j, SROWS)], dbuf)
        last = None
        for k in range(SROWS):
            if last is not None:
                last.wait()
            last = pltpu.async_copy(ones_v, acc.at[dbuf.at[k]], ssem,
                                    add=True)
        last.wait()
        return carry

    lax.fori_loop(0, NSTAGE, stage_body, 0)

    plsc.subcore_barrier()

    @pl.when(sid == 0)
    def _():
        pltpu.sync_copy(acc, out_hbm.at[cid])


_hist = functools.partial(
    pl.kernel,
    out_type=jax.ShapeDtypeStruct((2, NPAD), jnp.float32),
    mesh=_mesh,
    compiler_params=_SC_PARAMS,
    scratch_types=[
        pltpu.VMEM((SROWS, WIN), jnp.int32),
        pltpu.VMEM((WIN,), jnp.float32),
        pltpu.SemaphoreType.DMA,
        pltpu.VMEM_SHARED((NPAD,), jnp.float32),
    ],
)(_hist_body)


def _att0_body(src2d, dst2d, deg16, consts, zeros_n16, s_out,
               sbuf, dbuf, gs, gd, wbuf, cvm, gsem, ssem, acc_s):
    cid = lax.axis_index("c")
    sid = lax.axis_index("s")
    r0 = WROWS * (cid * 16 + sid)

    pltpu.sync_copy(consts, cvm)
    clv = cvm[0, :]
    crv = cvm[1, :]
    m0v = cvm[2, :]
    iota = lax.iota(jnp.int32, 16)
    tmask = ((iota >> 2) & 1) == 1

    @pl.when(sid == 0)
    def _():
        pltpu.sync_copy(zeros_n16, acc_s)
    plsc.subcore_barrier()

    def stage_body(j, carry):
        pltpu.sync_copy(src2d.at[pl.ds(r0 + SROWS * j, SROWS)], sbuf)
        pltpu.sync_copy(dst2d.at[pl.ds(r0 + SROWS * j, SROWS)], dbuf)
        gds = [(pltpu.async_copy(deg16.at[sbuf.at[k]], gs.at[k], gsem),
                pltpu.async_copy(deg16.at[dbuf.at[k]], gd.at[k], gsem))
               for k in range(SROWS)]
        last = None
        for k in range(SROWS):
            gds[k][0].wait()
            gds[k][1].wait()

            def row_block(rb, c2):
                for q in range(4):
                    r = 4 * rb + q
                    ds_ = gs[k, r, :]
                    dd_ = gd[k, r, :]
                    u = ds_ * clv + dd_ * crv
                    w = jnp.exp(jnp.maximum(u, NEG_SLOPE * u) - m0v)
                    wbuf[k, r, :] = jnp.where(tmask, w * ds_, w)
                return c2

            lax.fori_loop(0, 32, row_block, 0)
            if last is not None:
                last.wait()
            last = pltpu.async_copy(
                wbuf.at[k], acc_s.at[dbuf.at[k]], ssem, add=True)
        last.wait()
        return carry

    lax.fori_loop(0, NSTAGE, stage_body, 0)

    plsc.subcore_barrier()

    @pl.when(sid == 0)
    def _():
        pltpu.sync_copy(acc_s, s_out.at[cid])


_att0 = functools.partial(
    pl.kernel,
    out_type=jax.ShapeDtypeStruct((2, NPAD, 16), jnp.float32),
    mesh=_mesh,
    compiler_params=_SC_PARAMS,
    scratch_types=[
        pltpu.VMEM((SROWS, WIN), jnp.int32),
        pltpu.VMEM((SROWS, WIN), jnp.int32),
        pltpu.VMEM((SROWS, WIN, 16), jnp.float32),
        pltpu.VMEM((SROWS, WIN, 16), jnp.float32),
        pltpu.VMEM((SROWS, WIN, 16), jnp.float32),
        pltpu.VMEM((3, 16), jnp.float32),
        pltpu.SemaphoreType.DMA,
        pltpu.SemaphoreType.DMA,
        pltpu.VMEM_SHARED((NPAD, 16), jnp.float32),
    ],
)(_att0_body)


def _att_body(src2d, dst2d, el, er, mconst, zeros_n16, w_out, s_out,
              sbuf, dbuf, ges, ger, wbuf, wobuf, cvm, gsem, ssem, wsem,
              acc_s):
    cid = lax.axis_index("c")
    sid = lax.axis_index("s")
    r0 = WROWS * (cid * 16 + sid)

    pltpu.sync_copy(mconst, cvm)
    mv = cvm[0, :]

    @pl.when(sid == 0)
    def _():
        pltpu.sync_copy(zeros_n16, acc_s)
    plsc.subcore_barrier()

    def stage_body(j, carry):
        rr = r0 + SROWS * j
        pltpu.sync_copy(src2d.at[pl.ds(rr, SROWS)], sbuf)
        pltpu.sync_copy(dst2d.at[pl.ds(rr, SROWS)], dbuf)
        gds = [(pltpu.async_copy(el.at[sbuf.at[k]], ges.at[k], gsem),
                pltpu.async_copy(er.at[dbuf.at[k]], ger.at[k], gsem))
               for k in range(SROWS)]
        last = None
        wds = []
        for k in range(SROWS):
            gds[k][0].wait()
            gds[k][1].wait()

            def row_block(rb, c2):
                for q in range(4):
                    r = 4 * rb + q
                    u = ges[k, r, :] + ger[k, r, :]
                    w = jnp.exp(jnp.maximum(u, NEG_SLOPE * u) - mv)
                    wbuf[k, r, :] = w
                    wobuf[k, pl.ds(16 * r, 16)] = w
                return c2

            lax.fori_loop(0, 32, row_block, 0)
            if last is not None:
                last.wait()
            last = pltpu.async_copy(
                wbuf.at[k], acc_s.at[dbuf.at[k]], ssem, add=True)
            wds.append(pltpu.async_copy(
                wobuf.at[k], w_out.at[pl.ds((rr + k) * 2048, 2048)], wsem))
        last.wait()
        for d in wds:
            d.wait()
        return carry

    lax.fori_loop(0, NSTAGE, stage_body, 0)

    plsc.subcore_barrier()

    @pl.when(sid == 0)
    def _():
        pltpu.sync_copy(acc_s, s_out.at[cid])


_att = functools.partial(
    pl.kernel,
    out_type=(jax.ShapeDtypeStruct((EPAD * 16,), jnp.float32),
              jax.ShapeDtypeStruct((2, NPAD, 16), jnp.float32)),
    mesh=_mesh,
    compiler_params=_SC_PARAMS,
    scratch_types=[
        pltpu.VMEM((SROWS, WIN), jnp.int32),
        pltpu.VMEM((SROWS, WIN), jnp.int32),
        pltpu.VMEM((SROWS, WIN, 16), jnp.float32),
        pltpu.VMEM((SROWS, WIN, 16), jnp.float32),
        pltpu.VMEM((SROWS, WIN, 16), jnp.float32),
        pltpu.VMEM((SROWS, WIN * 16), jnp.float32),
        pltpu.VMEM((1, 16), jnp.float32),
        pltpu.SemaphoreType.DMA,
        pltpu.SemaphoreType.DMA,
        pltpu.SemaphoreType.DMA,
        pltpu.VMEM_SHARED((NPAD, 16), jnp.float32),
    ],
)(_att_body)


def _msg_body(src2d, dst2d, w_hbm, f0t, f1t, f2t, f3t, zeros_n16, out_hbm,
              sbuf, dbuf, wstg, fbuf, mbuf, gsem, ssem, acc):
    cid = lax.axis_index("c")
    sid = lax.axis_index("s")
    r0 = MROWS * sid

    def sweep(h, ftab):
        @pl.when(sid == 0)
        def _():
            pltpu.sync_copy(zeros_n16, acc)
        plsc.subcore_barrier()

        def stage_body(j, carry):
            rr = r0 + MSROWS * j
            pltpu.sync_copy(src2d.at[pl.ds(rr, MSROWS)], sbuf)
            pltpu.sync_copy(dst2d.at[pl.ds(rr, MSROWS)], dbuf)
            wd = pltpu.async_copy(
                w_hbm.at[pl.ds(rr * 2048, MSROWS * 2048)], wstg, gsem)
            gds = [pltpu.async_copy(ftab.at[sbuf.at[k]], fbuf.at[k], gsem)
                   for k in range(MSROWS)]
            wd.wait()
            last = None
            for k in range(MSROWS):
                gds[k].wait()

                def row_block(rb, c2):
                    for q in range(8):
                        r = 8 * rb + q
                        wv = wstg[pl.ds(2048 * k + 16 * r, 16)]
                        mbuf[k, r, :] = fbuf[k, r, :] * jnp.broadcast_to(
                            wv[h], (16,))
                    return c2

                lax.fori_loop(0, 16, row_block, 0)
                if last is not None:
                    last.wait()
                last = pltpu.async_copy(
                    mbuf.at[k], acc.at[dbuf.at[k]], ssem, add=True)
            last.wait()
            return carry

        lax.fori_loop(0, MSTAGE, stage_body, 0)
        plsc.subcore_barrier()

        @pl.when(sid == 0)
        def _():
            pltpu.sync_copy(acc, out_hbm.at[h])
        plsc.subcore_barrier()

    @pl.when(cid == 0)
    def _():
        sweep(0, f0t)
        sweep(1, f1t)

    @pl.when(cid == 1)
    def _():
        sweep(2, f2t)
        sweep(3, f3t)


_msg = functools.partial(
    pl.kernel,
    out_type=jax.ShapeDtypeStruct((4, NPAD, 16), jnp.float32),
    mesh=_mesh,
    compiler_params=_SC_PARAMS,
    scratch_types=[
        pltpu.VMEM((MSROWS, WIN), jnp.int32),
        pltpu.VMEM((MSROWS, WIN), jnp.int32),
        pltpu.VMEM((MSROWS * 2048,), jnp.float32),
        pltpu.VMEM((MSROWS, WIN, 16), jnp.float32),
        pltpu.VMEM((MSROWS, WIN, 16), jnp.float32),
        pltpu.SemaphoreType.DMA,
        pltpu.SemaphoreType.DMA,
        pltpu.VMEM_SHARED((NPAD, 16), jnp.float32),
    ],
)(_msg_body)


def _g_body(src2d, dst2d, w_hbm, s2t, zeros_n16, g_out,
            sbuf, dbuf, wstg, rbuf, abuf, gsem, ssem, acc_g):
    cid = lax.axis_index("c")
    sid = lax.axis_index("s")
    r0 = WROWS * (cid * 16 + sid)

    @pl.when(sid == 0)
    def _():
        pltpu.sync_copy(zeros_n16, acc_g)
    plsc.subcore_barrier()

    zero16 = jnp.zeros((16,), jnp.float32)

    def stage_body(j, carry):
        rr = r0 + SROWS * j
        pltpu.sync_copy(src2d.at[pl.ds(rr, SROWS)], sbuf)
        pltpu.sync_copy(dst2d.at[pl.ds(rr, SROWS)], dbuf)
        wd = pltpu.async_copy(
            w_hbm.at[pl.ds(rr * 2048, SROWS * 2048)], wstg, gsem)
        gds = [pltpu.async_copy(s2t.at[dbuf.at[k]], rbuf.at[k], gsem)
               for k in range(SROWS)]
        wd.wait()
        last = None
        for k in range(SROWS):
            gds[k].wait()

            def row_block(rb, c2):
                for q in range(4):
                    r = 4 * rb + q
                    sv = rbuf[k, r, :]
                    wv = wstg[pl.ds(2048 * k + 16 * r, 16)]
                    a = jnp.where(sv > 0, wv / jnp.maximum(sv, 1e-30),
                                  zero16)
                    abuf[k, r, :] = a
                return c2

            lax.fori_loop(0, 32, row_block, 0)
            if last is not None:
                last.wait()
            last = pltpu.async_copy(
                abuf.at[k], acc_g.at[sbuf.at[k]], ssem, add=True)
        last.wait()
        return carry

    lax.fori_loop(0, NSTAGE, stage_body, 0)

    plsc.subcore_barrier()

    @pl.when(sid == 0)
    def _():
        pltpu.sync_copy(acc_g, g_out.at[cid])


_g_pass = functools.partial(
    pl.kernel,
    out_type=jax.ShapeDtypeStruct((2, NPAD, 16), jnp.float32),
    mesh=_mesh,
    compiler_params=_SC_PARAMS,
    scratch_types=[
        pltpu.VMEM((SROWS, WIN), jnp.int32),
        pltpu.VMEM((SROWS, WIN), jnp.int32),
        pltpu.VMEM((SROWS * 2048,), jnp.float32),
        pltpu.VMEM((SROWS, WIN, 16), jnp.float32),
        pltpu.VMEM((SROWS, WIN, 16), jnp.float32),
        pltpu.SemaphoreType.DMA,
        pltpu.SemaphoreType.DMA,
        pltpu.VMEM_SHARED((NPAD, 16), jnp.float32),
    ],
)(_g_body)


def _t0_body(p_ref, deg16_ref, mm_ref):
    b = pl.program_id(0)
    deg = p_ref[0, :] + p_ref[1, :]
    deg16_ref[...] = jnp.broadcast_to(deg[:, None], (BN, 16))
    rows = b * BN + lax.iota(jnp.int32, BN)
    dmax = jnp.max(jnp.where(rows < N, deg, -jnp.inf))
    dmin = jnp.min(jnp.where(rows < N, deg, jnp.inf))

    @pl.when(b == 0)
    def _():
        mm_ref[...] = jnp.full((2, 128), 0.0, jnp.float32)
    mm_ref[0:1, :] = jnp.maximum(mm_ref[0:1, :], dmax)
    mm_ref[1:2, :] = jnp.minimum(mm_ref[1:2, :], dmin)


def _t0(degp):
    return pl.pallas_call(
        _t0_body,
        grid=(NBLK,),
        in_specs=[pl.BlockSpec((2, BN), lambda b: (0, b))],
        out_specs=[pl.BlockSpec((BN, 16), lambda b: (b, 0)),
                   pl.BlockSpec((2, 128), lambda b: (0, 0))],
        out_shape=[jax.ShapeDtypeStruct((NPAD, 16), jnp.float32),
                   jax.ShapeDtypeStruct((2, 128), jnp.float32)],
    )(degp)


def _t1_body(sp_ref, w0h_ref, w1c_ref, f0_ref, f1_ref, f2_ref, f3_ref,
             el_ref, er_ref, mm_ref):
    b = pl.program_id(0)
    s0 = sp_ref[0, :, 0:4] + sp_ref[1, :, 0:4]
    t0 = sp_ref[0, :, 4:8] + sp_ref[1, :, 4:8]
    S0 = jnp.where(s0 > 0, t0 / jnp.maximum(s0, 1e-30), 0.0)
    h1 = _elu(jnp.concatenate(
        [S0[:, h:h + 1] * w0h_ref[h:h + 1, :] for h in range(4)], axis=1))
    fe = jnp.dot(h1, w1c_ref[...], preferred_element_type=jnp.float32)
    f0_ref[...] = fe[:, 0:16]
    f1_ref[...] = fe[:, 16:32]
    f2_ref[...] = fe[:, 32:48]
    f3_ref[...] = fe[:, 48:64]
    el = fe[:, 64:68]
    er = fe[:, 68:72]
    el_ref[...] = _tile16(el)
    er_ref[...] = _tile16(er)
    mask = lax.broadcasted_iota(jnp.int32, (BN, 4), 0) + b * BN < N
    elm = jnp.max(jnp.where(mask, el, -jnp.inf), axis=0, keepdims=True)
    erm = jnp.max(jnp.where(mask, er, -jnp.inf), axis=0, keepdims=True)

    @pl.when(b == 0)
    def _():
        mm_ref[...] = jnp.full((2, 4), -jnp.inf, jnp.float32)
    mm_ref[0:1, :] = jnp.maximum(mm_ref[0:1, :], elm)
    mm_ref[1:2, :] = jnp.maximum(mm_ref[1:2, :], erm)


def _t1(sp, w0h, w1c):
    nspec = pl.BlockSpec((BN, 16), lambda b: (b, 0))
    nshape = jax.ShapeDtypeStruct((NPAD, 16), jnp.float32)
    return pl.pallas_call(
        _t1_body,
        grid=(NBLK,),
        in_specs=[pl.BlockSpec((2, BN, 16), lambda b: (0, b, 0)),
                  pl.BlockSpec((4, 16), lambda b: (0, 0)),
                  pl.BlockSpec((64, 72), lambda b: (0, 0))],
        out_specs=[nspec, nspec, nspec, nspec, nspec, nspec,
                   pl.BlockSpec((2, 4), lambda b: (0, 0))],
        out_shape=[nshape, nshape, nshape, nshape, nshape, nshape,
                   jax.ShapeDtypeStruct((2, 4), jnp.float32)],
    )(sp, w0h, w1c)


def _t2_body(o_ref, s_ref, b2_ref, h2_ref, el_ref, er_ref, mm_ref):
    b = pl.program_id(0)
    s1 = s_ref[0, :, 0:4] + s_ref[1, :, 0:4]
    parts = []
    for h in range(4):
        oh = o_ref[h]
        sh = s1[:, h:h + 1]
        parts.append(jnp.where(sh > 0, oh / jnp.maximum(sh, 1e-30), 0.0))
    h2 = _elu(jnp.concatenate(parts, axis=1))
    h2_ref[...] = h2
    ee = jnp.dot(h2, b2_ref[...], preferred_element_type=jnp.float32)
    el = ee[:, 0:4]
    er = ee[:, 4:8]
    el_ref[...] = _tile16(el)
    er_ref[...] = _tile16(er)
    mask = lax.broadcasted_iota(jnp.int32, (BN, 4), 0) + b * BN < N
    elm = jnp.max(jnp.where(mask, el, -jnp.inf), axis=0, keepdims=True)
    erm = jnp.max(jnp.where(mask, er, -jnp.inf), axis=0, keepdims=True)

    @pl.when(b == 0)
    def _():
        mm_ref[...] = jnp.full((2, 4), -jnp.inf, jnp.float32)
    mm_ref[0:1, :] = jnp.maximum(mm_ref[0:1, :], elm)
    mm_ref[1:2, :] = jnp.maximum(mm_ref[1:2, :], erm)


def _t2(out1, s1p, b2):
    return pl.pallas_call(
        _t2_body,
        grid=(NBLK,),
        in_specs=[pl.BlockSpec((4, BN, 16), lambda b: (0, b, 0)),
                  pl.BlockSpec((2, BN, 16), lambda b: (0, b, 0)),
                  pl.BlockSpec((64, 8), lambda b: (0, 0))],
        out_specs=[pl.BlockSpec((BN, 64), lambda b: (b, 0)),
                   pl.BlockSpec((BN, 16), lambda b: (b, 0)),
                   pl.BlockSpec((BN, 16), lambda b: (b, 0)),
                   pl.BlockSpec((2, 4), lambda b: (0, 0))],
        out_shape=[jax.ShapeDtypeStruct((NPAD, 64), jnp.float32),
                   jax.ShapeDtypeStruct((NPAD, 16), jnp.float32),
                   jax.ShapeDtypeStruct((NPAD, 16), jnp.float32),
                   jax.ShapeDtypeStruct((2, 4), jnp.float32)],
    )(out1, s1p, b2)


def _ts_body(s_ref, st_ref):
    s2 = s_ref[0, :, 0:4] + s_ref[1, :, 0:4]
    st_ref[...] = _tile16(s2)


def _ts(s2p):
    return pl.pallas_call(
        _ts_body,
        grid=(NBLK,),
        in_specs=[pl.BlockSpec((2, BN, 16), lambda b: (0, b, 0))],
        out_specs=pl.BlockSpec((BN, 16), lambda b: (b, 0)),
        out_shape=jax.ShapeDtypeStruct((NPAD, 16), jnp.float32),
    )(s2p)


def _t3_body(g_ref, h2_ref, G_ref):
    b = pl.program_id(0)
    g = g_ref[0, :, 0:4] + g_ref[1, :, 0:4]
    mask = lax.broadcasted_iota(jnp.int32, (BN, 4), 0) + b * BN < N
    g = jnp.where(mask, g, 0.0)
    Gb = lax.dot_general(g, h2_ref[...], (((0,), (0,)), ((), ())),
                         preferred_element_type=jnp.float32)

    @pl.when(b == 0)
    def _():
        G_ref[...] = jnp.zeros((4, 64), jnp.float32)
    G_ref[...] += Gb


def _t3(gp, h2):
    return pl.pallas_call(
        _t3_body,
        grid=(NBLK,),
        in_specs=[pl.BlockSpec((2, BN, 16), lambda b: (0, b, 0)),
                  pl.BlockSpec((BN, 64), lambda b: (b, 0))],
        out_specs=pl.BlockSpec((4, 64), lambda b: (0, 0)),
        out_shape=jax.ShapeDtypeStruct((4, 64), jnp.float32),
    )(gp, h2)


def _pad_edges(idx):
    return jnp.concatenate(
        [idx, jnp.full((EPAD - E,), N, jnp.int32)]).reshape(ROWS, WIN)


def kernel(edge_index, W0, a_l0, a_r0, W1, a_l1, a_r1, W2, a_l2, a_r2):
    src2d = _pad_edges(edge_index[0].astype(jnp.int32))
    dst2d = _pad_edges(edge_index[1].astype(jnp.int32))
    zeros_n = jnp.zeros((NPAD,), jnp.float32)
    zeros_n16 = jnp.zeros((NPAD, 16), jnp.float32)

    degp = _hist(dst2d, zeros_n)
    deg16, dmm = _t0(degp)
    dmax, dmin = dmm[0, 0], dmm[1, 0]

    W0h = W0.reshape(H, D_HID)
    cl0 = jnp.sum(W0h * a_l0, axis=1)
    cr0 = jnp.sum(W0h * a_r0, axis=1)
    mel0 = jnp.maximum(cl0 * dmax, cl0 * dmin)
    mer0 = jnp.maximum(cr0 * dmax, cr0 * dmin)
    mh0 = _lr(mel0 + mer0)
    consts0 = jnp.stack([jnp.tile(cl0, 4), jnp.tile(cr0, 4),
                         jnp.tile(mh0, 4)])
    sp0 = _att0(src2d, dst2d, deg16, consts0, zeros_n16)

    W1r = W1.reshape(64, H, D_HID)
    bl1 = jnp.einsum("khd,hd->kh", W1r, a_l1)
    br1 = jnp.einsum("khd,hd->kh", W1r, a_r1)
    w1c = jnp.concatenate([W1, bl1, br1], axis=1)
    f0t, f1t, f2t, f3t, el1, er1, mm1 = _t1(sp0, W0h, w1c)
    mh1 = _lr(mm1[0] + mm1[1])
    w1, s1p = _att(src2d, dst2d, el1, er1,
                   jnp.tile(mh1, 4)[None, :], zeros_n16)
    out1 = _msg(src2d, dst2d, w1, f0t, f1t, f2t, f3t,
                zeros_n16)

    W2r = W2.reshape(64, H, N_CLASSES)
    bl2 = jnp.einsum("khc,hc->kh", W2r, a_l2)
    br2 = jnp.einsum("khc,hc->kh", W2r, a_r2)
    b2 = jnp.concatenate([bl2, br2], axis=1)
    h2, el2, er2, mm2 = _t2(out1, s1p, b2)
    mh2 = _lr(mm2[0] + mm2[1])
    w2, s2p = _att(src2d, dst2d, el2, er2,
                   jnp.tile(mh2, 4)[None, :], zeros_n16)
    s2t = _ts(s2p)
    gp = _g_pass(src2d, dst2d, w2, s2t, zeros_n16)

    G = _t3(gp, h2)
    hg = jnp.einsum("hk,khc->c", G, W2r) / (N * H)
    return hg[None, :]
